# baseline (speedup 1.0000x reference)
import functools

import jax
import jax.numpy as jnp
from jax import lax
from jax.experimental import pallas as pl
from jax.experimental.pallas import tpu as pltpu
from jax.experimental.pallas import tpu_sc as plsc

BATCH = 16384
EMBED_DIM = 32
CHUNK = 128


def _make_kernel(num_cores, num_workers, b_per_w):
    mesh = plsc.VectorSubcoreMesh(core_axis_name="c", subcore_axis_name="s")
    n_chunks = b_per_w // CHUNK

    @functools.partial(
        pl.kernel,
        mesh=mesh,
        compiler_params=pltpu.CompilerParams(
            needs_layout_passes=False, use_tc_tiling_on_sc=False
        ),
        out_type=jax.ShapeDtypeStruct((BATCH,), jnp.float32),
        scratch_types=[
            pltpu.VMEM((b_per_w,), jnp.int32),
            pltpu.VMEM((b_per_w,), jnp.int32),
            pltpu.VMEM((b_per_w, EMBED_DIM), jnp.float32),
            pltpu.VMEM((b_per_w, EMBED_DIM), jnp.float32),
            pltpu.VMEM((b_per_w,), jnp.float32),
            pltpu.VMEM((b_per_w,), jnp.float32),
            pltpu.VMEM((b_per_w,), jnp.float32),
            pltpu.SemaphoreType.DMA,
        ],
    )
    def sc_kernel(i_hbm, j_hbm, m_hbm, d_hbm, u_hbm, v_hbm, out_hbm,
                  idx_i, idx_j, u_rows, v_rows, m_v, d_v, out_v, sem):
        wid = lax.axis_index("s") * num_cores + lax.axis_index("c")
        base = wid * b_per_w

        pltpu.sync_copy(i_hbm.at[pl.ds(base, b_per_w)], idx_i)
        pltpu.sync_copy(j_hbm.at[pl.ds(base, b_per_w)], idx_j)

        copies = []
        for c in range(n_chunks):
            s = pl.ds(c * CHUNK, CHUNK)
            copies.append(pltpu.async_copy(u_hbm.at[idx_i.at[s]], u_rows.at[s], sem))
            copies.append(pltpu.async_copy(v_hbm.at[idx_j.at[s]], v_rows.at[s], sem))
            copies.append(pltpu.async_copy(m_hbm.at[idx_i.at[s]], m_v.at[s], sem))
            copies.append(pltpu.async_copy(d_hbm.at[idx_j.at[s]], d_v.at[s], sem))
        for cp in copies:
            cp.wait()

        lane = lax.iota(jnp.int32, 16)

        def group_body(g, carry):
            gb = g * 16
            p_vec = gb + lane
            acc = m_v[pl.ds(gb, 16)] + d_v[pl.ds(gb, 16)]
            for k in range(EMBED_DIM):
                k_vec = jnp.full((16,), k, dtype=jnp.int32)
                uk = plsc.load_gather(u_rows, [p_vec, k_vec])
                vk = plsc.load_gather(v_rows, [p_vec, k_vec])
                acc = acc + uk * vk
            out_v[pl.ds(gb, 16)] = acc
            return carry

        lax.fori_loop(0, b_per_w // 16, group_body, 0)

        pltpu.sync_copy(out_v, out_hbm.at[pl.ds(base, b_per_w)])

    return sc_kernel


def kernel(ij, m_bar, d_bar, U, V):
    i = jnp.asarray(ij[:, 0], dtype=jnp.int32)
    j = jnp.asarray(ij[:, 1], dtype=jnp.int32)
    info = plsc.get_sparse_core_info()
    num_workers = info.num_cores * info.num_subcores
    b_per_w = BATCH // num_workers
    return _make_kernel(info.num_cores, num_workers, b_per_w)(i, j, m_bar, d_bar, U, V)
